# baseline (speedup 1.0000x reference)
import functools

import jax
import jax.numpy as jnp
from jax.experimental import pallas as pl
from jax.experimental.pallas import tpu as pltpu


def _ccp_kernel(cu_row_ref, cu_col_ref, flat_a_ref, flat_b_ref, W_pz_ref,
                W_lik_ref, b_pz_ref, b_lik_ref, out_ref, V_s, acc_seg,
                *, B, T, E, Z, NSEG, nb):
    i = pl.program_id(0)

    @pl.when(i == 0)
    def _init():
        w2 = W_lik_ref[E:E + Z, :]
        va = jax.lax.dot_general(
            W_pz_ref[0:E, 0:Z], w2, (((1,), (0,)), ((), ())),
            preferred_element_type=jnp.float32) + W_lik_ref[E + Z:2 * E + Z, :]
        vu = jax.lax.dot_general(
            W_pz_ref[E:2 * E, 0:Z], w2, (((1,), (0,)), ((), ())),
            preferred_element_type=jnp.float32)
        vg = jax.lax.dot_general(
            W_pz_ref[2 * E:3 * E, 0:Z], w2, (((1,), (0,)), ((), ())),
            preferred_element_type=jnp.float32) + W_lik_ref[2 * E + Z:3 * E + Z, :]
        w1 = W_lik_ref[0:E, :]
        V_s[:, :] = jnp.concatenate(
            [w1, vu, vg, va, jnp.zeros((E, 4), jnp.float32)], axis=1)
        acc_seg[:, :] = jnp.zeros((2 * NSEG, 8), jnp.float32)

    starts_c = jnp.concatenate(
        [cu_col_ref[0:NSEG, :], cu_col_ref[0:NSEG, :]], axis=0)
    ends_c = jnp.concatenate(
        [cu_col_ref[1:NSEG + 1, :], cu_col_ref[0:NSEG, :] + 1], axis=0)
    acc = acc_seg[:, :]
    for half, ref in ((0, flat_a_ref), (1, flat_b_ref)):
        P = jax.lax.dot_general(ref[:, :], V_s[:, :],
                                (((1,), (0,)), ((), ())),
                                preferred_element_type=jnp.float32)
        base = (2 * i + half) * B
        pos = jax.lax.broadcasted_iota(jnp.int32, (2 * NSEG, B), 1) + base
        m = ((pos >= starts_c) & (pos < ends_c)).astype(jnp.float32)
        acc = acc + jax.lax.dot_general(
            m, P, (((1,), (0,)), ((), ())), preferred_element_type=jnp.float32)
        out_ref[pl.ds((2 * i + half) * (B // 128), B // 128), :] = (
            P[:, 0:1].reshape(B // 128, 128))
    acc_seg[:, :] = acc

    @pl.when(i == nb - 1)
    def _finish():
        startsf = cu_col_ref[0:NSEG, :].astype(jnp.float32)
        endsf = cu_col_ref[1:NSEG + 1, :].astype(jnp.float32)
        counts = endsf - startsf
        SU = acc_seg[0:NSEG, 1:2]
        SG = acc_seg[0:NSEG, 2:3]
        u_anc = acc_seg[NSEG:2 * NSEG, 1:2]
        a_anc = acc_seg[NSEG:2 * NSEG, 3:4]
        w2 = W_lik_ref[E:E + Z, :]
        cval = jax.lax.dot_general(
            b_pz_ref[:, 0:Z], w2, (((1,), (0,)), ((), ())),
            preferred_element_type=jnp.float32) + b_lik_ref[:, :]
        s = a_anc + (SU - u_anc) / (counts - 1.0) + SG / counts + cval
        pos = (jax.lax.broadcasted_iota(jnp.int32, (T // 128, 128), 0) * 128
               + jax.lax.broadcasted_iota(jnp.int32, (T // 128, 128), 1))
        s_tok = jnp.zeros((T // 128, 128), jnp.float32)
        prev = jnp.zeros((), jnp.float32)
        for b in range(NSEG):
            db = s[b, 0] - prev
            prev = s[b, 0]
            s_tok = s_tok + jnp.where(pos >= cu_col_ref[b, 0], db, 0.0)
        out_ref[:, :] = out_ref[:, :] + s_tok


@functools.partial(jax.jit, static_argnames=("interpret",))
def _run(flat, cu_seqlens, W_pz, b_pz, W_lik, b_lik, interpret=False):
    T, E = flat.shape
    Z = W_pz.shape[1] // 2
    NSEG = cu_seqlens.shape[0] - 1
    B = 4096
    nb = T // (2 * B)

    cu_pad = jnp.zeros((2 * NSEG,), jnp.int32).at[:NSEG + 1].set(cu_seqlens)
    cu_row = cu_pad.reshape(1, 2 * NSEG)
    cu_col = cu_pad.reshape(2 * NSEG, 1)
    b_pz_row = b_pz.reshape(1, -1)
    b_lik_2d = b_lik.reshape(1, 1)

    kern = functools.partial(_ccp_kernel, B=B, T=T, E=E, Z=Z,
                             NSEG=NSEG, nb=nb)
    out = pl.pallas_call(
        kern,
        grid=(nb,),
        in_specs=[
            pl.BlockSpec((1, 2 * NSEG), lambda i: (0, 0)),
            pl.BlockSpec((2 * NSEG, 1), lambda i: (0, 0)),
            pl.BlockSpec((B, E), lambda i: (2 * i, 0)),
            pl.BlockSpec((B, E), lambda i: (2 * i + 1, 0)),
            pl.BlockSpec((3 * E, 2 * Z), lambda i: (0, 0)),
            pl.BlockSpec((3 * E + Z, 1), lambda i: (0, 0)),
            pl.BlockSpec((1, 2 * Z), lambda i: (0, 0)),
            pl.BlockSpec((1, 1), lambda i: (0, 0)),
        ],
        out_specs=pl.BlockSpec((T // 128, 128), lambda i: (0, 0)),
        out_shape=jax.ShapeDtypeStruct((T // 128, 128), jnp.float32),
        scratch_shapes=[
            pltpu.VMEM((E, 8), jnp.float32),
            pltpu.VMEM((2 * NSEG, 8), jnp.float32),
        ],
        interpret=interpret,
    )(cu_row, cu_col, flat, flat, W_pz, W_lik, b_pz_row, b_lik_2d)
    return out.reshape(T)


def kernel(flat, cu_seqlens, W_pz, b_pz, W_lik, b_lik):
    return _run(flat, cu_seqlens, W_pz, b_pz, W_lik, b_lik)


# speedup vs baseline: 1.1360x; 1.1360x over previous
import functools

import jax
import jax.numpy as jnp
from jax.experimental import pallas as pl
from jax.experimental.pallas import tpu as pltpu


def _ccp_kernel(cu_row_ref, cu_col_ref, flat_ref, W_pz_ref, W_lik_ref,
                b_pz_ref, b_lik_ref, out_ref, V_s, acc_seg,
                *, B, T, E, Z, NSEG, nb):
    i = pl.program_id(0)

    @pl.when(i == 0)
    def _init():
        w2 = W_lik_ref[E:E + Z, :]
        va = jax.lax.dot_general(
            W_pz_ref[0:E, 0:Z], w2, (((1,), (0,)), ((), ())),
            preferred_element_type=jnp.float32) + W_lik_ref[E + Z:2 * E + Z, :]
        vu = jax.lax.dot_general(
            W_pz_ref[E:2 * E, 0:Z], w2, (((1,), (0,)), ((), ())),
            preferred_element_type=jnp.float32)
        vg = jax.lax.dot_general(
            W_pz_ref[2 * E:3 * E, 0:Z], w2, (((1,), (0,)), ((), ())),
            preferred_element_type=jnp.float32) + W_lik_ref[2 * E + Z:3 * E + Z, :]
        w1 = W_lik_ref[0:E, :]
        V_s[:, :] = jnp.concatenate(
            [w1, vu, vg, va, jnp.zeros((E, 4), jnp.float32)], axis=1)
        acc_seg[:, :] = jnp.zeros((2 * NSEG, 8), jnp.float32)

    P = jax.lax.dot_general(flat_ref[:, :], V_s[:, :],
                            (((1,), (0,)), ((), ())),
                            preferred_element_type=jnp.float32)

    pos = jax.lax.broadcasted_iota(jnp.int32, (2 * NSEG, B), 1) + i * B
    starts_c = jnp.concatenate(
        [cu_col_ref[0:NSEG, :], cu_col_ref[0:NSEG, :]], axis=0)
    ends_c = jnp.concatenate(
        [cu_col_ref[1:NSEG + 1, :], cu_col_ref[0:NSEG, :] + 1], axis=0)
    m = ((pos >= starts_c) & (pos < ends_c)).astype(jnp.float32)
    acc_seg[:, :] += jax.lax.dot_general(
        m, P, (((1,), (0,)), ((), ())), preferred_element_type=jnp.float32)

    out_ref[pl.ds(i * (B // 128), B // 128), :] = P[:, 0:1].reshape(B // 128, 128)

    @pl.when(i == nb - 1)
    def _finish():
        startsf = cu_col_ref[0:NSEG, :].astype(jnp.float32)
        endsf = cu_col_ref[1:NSEG + 1, :].astype(jnp.float32)
        counts = endsf - startsf
        SU = acc_seg[0:NSEG, 1:2]
        SG = acc_seg[0:NSEG, 2:3]
        u_anc = acc_seg[NSEG:2 * NSEG, 1:2]
        a_anc = acc_seg[NSEG:2 * NSEG, 3:4]
        w2 = W_lik_ref[E:E + Z, :]
        cval = jax.lax.dot_general(
            b_pz_ref[:, 0:Z], w2, (((1,), (0,)), ((), ())),
            preferred_element_type=jnp.float32) + b_lik_ref[:, :]
        s = a_anc + (SU - u_anc) / (counts - 1.0) + SG / counts + cval
        pos = (jax.lax.broadcasted_iota(jnp.int32, (T // 128, 128), 0) * 128
               + jax.lax.broadcasted_iota(jnp.int32, (T // 128, 128), 1))
        s_tok = jnp.zeros((T // 128, 128), jnp.float32)
        prev = jnp.zeros((), jnp.float32)
        for b in range(NSEG):
            db = s[b, 0] - prev
            prev = s[b, 0]
            s_tok = s_tok + jnp.where(pos >= cu_col_ref[b, 0], db, 0.0)
        out_ref[:, :] = out_ref[:, :] + s_tok


@functools.partial(jax.jit, static_argnames=("interpret",))
def _run(flat, cu_seqlens, W_pz, b_pz, W_lik, b_lik, interpret=False):
    T, E = flat.shape
    Z = W_pz.shape[1] // 2
    NSEG = cu_seqlens.shape[0] - 1
    B = 8192
    nb = T // B

    cu_pad = jnp.zeros((2 * NSEG,), jnp.int32).at[:NSEG + 1].set(cu_seqlens)
    cu_row = cu_pad.reshape(1, 2 * NSEG)
    cu_col = cu_pad.reshape(2 * NSEG, 1)
    b_pz_row = b_pz.reshape(1, -1)
    b_lik_2d = b_lik.reshape(1, 1)

    kern = functools.partial(_ccp_kernel, B=B, T=T, E=E, Z=Z,
                             NSEG=NSEG, nb=nb)
    out = pl.pallas_call(
        kern,
        grid=(nb,),
        in_specs=[
            pl.BlockSpec((1, 2 * NSEG), lambda i: (0, 0)),
            pl.BlockSpec((2 * NSEG, 1), lambda i: (0, 0)),
            pl.BlockSpec((B, E), lambda i: (i, 0)),
            pl.BlockSpec((3 * E, 2 * Z), lambda i: (0, 0)),
            pl.BlockSpec((3 * E + Z, 1), lambda i: (0, 0)),
            pl.BlockSpec((1, 2 * Z), lambda i: (0, 0)),
            pl.BlockSpec((1, 1), lambda i: (0, 0)),
        ],
        out_specs=pl.BlockSpec((T // 128, 128), lambda i: (0, 0)),
        out_shape=jax.ShapeDtypeStruct((T // 128, 128), jnp.float32),
        scratch_shapes=[
            pltpu.VMEM((E, 8), jnp.float32),
            pltpu.VMEM((2 * NSEG, 8), jnp.float32),
        ],
        interpret=interpret,
    )(cu_row, cu_col, flat, W_pz, W_lik, b_pz_row, b_lik_2d)
    return out.reshape(T)


def kernel(flat, cu_seqlens, W_pz, b_pz, W_lik, b_lik):
    return _run(flat, cu_seqlens, W_pz, b_pz, W_lik, b_lik)
